# Initial kernel scaffold; baseline (speedup 1.0000x reference)
import functools

import jax
import jax.numpy as jnp
from jax import lax
from jax.experimental import pallas as pl
from jax.experimental.pallas import tpu as pltpu
from jax.experimental.pallas import tpu_sc as plsc

_N = 2048
_NNZ = 32768
_LAYERS = 2

_NC = 2
_NS = 16
_EPT = _NNZ // _NS
_ROWS_PER_BAND = 512
_BANDS_PER_CORE = 2
_BAND_WORDS = _ROWS_PER_BAND * _N
_SLICE_W = _BAND_WORDS // _NS
_ZERO_W = 32768
_CHUNK = 128


def _densify_body(rows_hbm, cols_hbm, va_hbm, vw_hbm, outa_hbm, outw_hbm,
                  rows_v, cols_v, va_v, vw_v, off_v, val_v, zero_v, band_sp):
    c = lax.axis_index("c")
    s = lax.axis_index("s")
    base = s * _EPT
    pltpu.sync_copy(rows_hbm.at[pl.ds(base, _EPT)], rows_v)
    pltpu.sync_copy(cols_hbm.at[pl.ds(base, _EPT)], cols_v)
    pltpu.sync_copy(va_hbm.at[pl.ds(base, _EPT)], va_v)
    pltpu.sync_copy(vw_hbm.at[pl.ds(base, _EPT)], vw_v)

    def zfill(i, _):
        zero_v[pl.ds(i * 16, 16)] = jnp.zeros((16,), jnp.float32)
        return 0
    lax.fori_loop(0, _ZERO_W // 16, zfill, 0)

    for vals_v, out_hbm in ((va_v, outa_hbm), (vw_v, outw_hbm)):
        for band_i in range(_BANDS_PER_CORE):
            band = c * _BANDS_PER_CORE + band_i

            for z in range(_SLICE_W // _ZERO_W):
                pltpu.sync_copy(
                    zero_v,
                    band_sp.at[pl.ds(s * _SLICE_W + z * _ZERO_W, _ZERO_W)])

            def compute(t, _):
                row = t // (_CHUNK // 16)
                k = t % (_CHUNK // 16)
                r = rows_v[pl.ds(t * 16, 16)]
                cc = cols_v[pl.ds(t * 16, 16)]
                v = vals_v[pl.ds(t * 16, 16)]
                inb = lax.shift_right_logical(r, 9) == band
                off = jnp.where(
                    inb, lax.shift_left(jnp.bitwise_and(r, 511), 11) + cc, 0)
                vv = jnp.where(inb, v, 0.0)
                off_v[row, pl.ds(k * 16, 16)] = off
                val_v[row, pl.ds(k * 16, 16)] = vv
                return 0
            lax.fori_loop(0, _EPT // 16, compute, 0)

            plsc.subcore_barrier()

            def scatter(j, _):
                pltpu.sync_copy(val_v.at[j], band_sp.at[off_v.at[j]], add=True)
                return 0
            lax.fori_loop(0, _EPT // _CHUNK, scatter, 0)

            plsc.subcore_barrier()

            pltpu.sync_copy(
                band_sp.at[pl.ds(s * _SLICE_W, _SLICE_W)],
                out_hbm.at[pl.ds(band * _BAND_WORDS + s * _SLICE_W, _SLICE_W)])

            plsc.subcore_barrier()


_densify = pl.kernel(
    _densify_body,
    out_type=[jax.ShapeDtypeStruct((_N * _N,), jnp.float32),
              jax.ShapeDtypeStruct((_N * _N,), jnp.float32)],
    mesh=plsc.VectorSubcoreMesh(core_axis_name="c", subcore_axis_name="s"),
    scratch_types=[
        pltpu.VMEM((_EPT,), jnp.int32),
        pltpu.VMEM((_EPT,), jnp.int32),
        pltpu.VMEM((_EPT,), jnp.float32),
        pltpu.VMEM((_EPT,), jnp.float32),
        pltpu.VMEM((_EPT // _CHUNK, _CHUNK), jnp.int32),
        pltpu.VMEM((_EPT // _CHUNK, _CHUNK), jnp.float32),
        pltpu.VMEM((_ZERO_W,), jnp.float32),
        pltpu.VMEM_SHARED((_BAND_WORDS,), jnp.float32),
    ],
)


_BM = 512
_BN = 512


def _mm_body(a_ref, b_ref, bias_ref, o_ref):
    o_ref[...] = jnp.dot(a_ref[...], b_ref[...],
                         preferred_element_type=jnp.float32) + bias_ref[...]


def _mm(a, b, bias_row):
    grid = (_N // _BM, _N // _BN)
    return pl.pallas_call(
        _mm_body,
        grid=grid,
        in_specs=[
            pl.BlockSpec((_BM, _N), lambda i, j: (i, 0)),
            pl.BlockSpec((_N, _BN), lambda i, j: (0, j)),
            pl.BlockSpec((1, _BN), lambda i, j: (0, j)),
        ],
        out_specs=pl.BlockSpec((_BM, _BN), lambda i, j: (i, j)),
        out_shape=jax.ShapeDtypeStruct((_N, _N), jnp.float32),
    )(a, b, bias_row)


def kernel(x, rows, cols, adj_vals, W_vals, bias):
    ad_flat, wd_flat = _densify(rows, cols, adj_vals, W_vals)
    a_d = ad_flat.reshape(_N, _N)
    w_d = wd_flat.reshape(_N, _N)
    zero_row = jnp.zeros((1, _N), jnp.float32)
    bias_row = bias.reshape(1, _N)
    m = _mm(a_d, w_d, zero_row)
    h = x
    for _ in range(_LAYERS):
        h = _mm(m, h, bias_row)
    return h


# speedup vs baseline: 9.7940x; 9.7940x over previous
import functools

import jax
import jax.numpy as jnp
from jax import lax
from jax.experimental import pallas as pl
from jax.experimental.pallas import tpu as pltpu
from jax.experimental.pallas import tpu_sc as plsc

_N = 2048
_NNZ = 32768
_LAYERS = 2

_NC = 2
_NS = 16
_EPT = _NNZ // _NS
_ROWS_PER_BAND = 512
_BANDS_PER_CORE = 2
_BAND_WORDS = _ROWS_PER_BAND * _N
_SLICE_W = _BAND_WORDS // _NS
_ZERO_W = 32768
_CHUNK = 128


def _densify_body(rows_hbm, cols_hbm, va_hbm, vw_hbm, outa_hbm, outw_hbm,
                  rows_v, cols_v, va_v, vw_v, off_v, val_v, zero_v, band_sp):
    c = lax.axis_index("c")
    s = lax.axis_index("s")
    base = s * _EPT
    pltpu.sync_copy(rows_hbm.at[pl.ds(base, _EPT)], rows_v)
    pltpu.sync_copy(cols_hbm.at[pl.ds(base, _EPT)], cols_v)
    pltpu.sync_copy(va_hbm.at[pl.ds(base, _EPT)], va_v)
    pltpu.sync_copy(vw_hbm.at[pl.ds(base, _EPT)], vw_v)

    def zfill(i, _):
        zero_v[pl.ds(i * 16, 16)] = jnp.zeros((16,), jnp.float32)
        return 0
    lax.fori_loop(0, _ZERO_W // 16, zfill, 0)

    for vals_v, out_hbm in ((va_v, outa_hbm), (vw_v, outw_hbm)):
        for band_i in range(_BANDS_PER_CORE):
            band = c * _BANDS_PER_CORE + band_i

            for z in range(_SLICE_W // _ZERO_W):
                pltpu.sync_copy(
                    zero_v,
                    band_sp.at[pl.ds(s * _SLICE_W + z * _ZERO_W, _ZERO_W)])

            def compute(t, _):
                row = t // (_CHUNK // 16)
                k = t % (_CHUNK // 16)
                r = rows_v[pl.ds(t * 16, 16)]
                cc = cols_v[pl.ds(t * 16, 16)]
                v = vals_v[pl.ds(t * 16, 16)]
                inb = lax.shift_right_logical(r, 9) == band
                off = jnp.where(
                    inb, lax.shift_left(jnp.bitwise_and(r, 511), 11) + cc, 0)
                vv = jnp.where(inb, v, 0.0)
                off_v[row, pl.ds(k * 16, 16)] = off
                val_v[row, pl.ds(k * 16, 16)] = vv
                return 0
            lax.fori_loop(0, _EPT // 16, compute, 0)

            plsc.subcore_barrier()

            def scatter_round(rnd, _):
                @pl.when(s == rnd)
                def _inner():
                    def scatter(j, _):
                        pltpu.sync_copy(val_v.at[j],
                                        band_sp.at[off_v.at[j]], add=True)
                        return 0
                    lax.fori_loop(0, _EPT // _CHUNK, scatter, 0)
                plsc.subcore_barrier()
                return 0
            lax.fori_loop(0, _NS, scatter_round, 0)

            plsc.subcore_barrier()

            pltpu.sync_copy(
                band_sp.at[pl.ds(s * _SLICE_W, _SLICE_W)],
                out_hbm.at[pl.ds(band * _BAND_WORDS + s * _SLICE_W, _SLICE_W)])

            plsc.subcore_barrier()


_densify = pl.kernel(
    _densify_body,
    out_type=[jax.ShapeDtypeStruct((_N * _N,), jnp.float32),
              jax.ShapeDtypeStruct((_N * _N,), jnp.float32)],
    mesh=plsc.VectorSubcoreMesh(core_axis_name="c", subcore_axis_name="s"),
    scratch_types=[
        pltpu.VMEM((_EPT,), jnp.int32),
        pltpu.VMEM((_EPT,), jnp.int32),
        pltpu.VMEM((_EPT,), jnp.float32),
        pltpu.VMEM((_EPT,), jnp.float32),
        pltpu.VMEM((_EPT // _CHUNK, _CHUNK), jnp.int32),
        pltpu.VMEM((_EPT // _CHUNK, _CHUNK), jnp.float32),
        pltpu.VMEM((_ZERO_W,), jnp.float32),
        pltpu.VMEM_SHARED((_BAND_WORDS,), jnp.float32),
    ],
)


_BM = 512
_BN = 512


def _mm_body(a_ref, b_ref, bias_ref, o_ref):
    o_ref[...] = jnp.dot(a_ref[...], b_ref[...],
                         preferred_element_type=jnp.float32) + bias_ref[...]


def _mm(a, b, bias_row):
    grid = (_N // _BM, _N // _BN)
    return pl.pallas_call(
        _mm_body,
        grid=grid,
        in_specs=[
            pl.BlockSpec((_BM, _N), lambda i, j: (i, 0)),
            pl.BlockSpec((_N, _BN), lambda i, j: (0, j)),
            pl.BlockSpec((1, _BN), lambda i, j: (0, j)),
        ],
        out_specs=pl.BlockSpec((_BM, _BN), lambda i, j: (i, j)),
        out_shape=jax.ShapeDtypeStruct((_N, _N), jnp.float32),
    )(a, b, bias_row)


def kernel(x, rows, cols, adj_vals, W_vals, bias):
    ad_flat, wd_flat = _densify(rows, cols, adj_vals, W_vals)
    a_d = ad_flat.reshape(_N, _N)
    w_d = wd_flat.reshape(_N, _N)
    zero_row = jnp.zeros((1, _N), jnp.float32)
    bias_row = bias.reshape(1, _N)
    m = _mm(a_d, w_d, zero_row)
    h = x
    for _ in range(_LAYERS):
        h = _mm(m, h, bias_row)
    return h
